# Initial kernel scaffold; baseline (speedup 1.0000x reference)
import jax
import jax.numpy as jnp
from jax.experimental import pallas as pl


def kernel(x, W1, b1, W2, b2):
    raise NotImplementedError("write your pallas kernel here")


# speedup vs baseline: 8.6633x; 8.6633x over previous
import functools

import jax
import jax.numpy as jnp
from jax.experimental import pallas as pl

MIN_S = 0.05
MAX_S = 0.3
B = 8192
D = 4096
H = D // 4
BR = 128
NBLK = B // BR
SEARCH_ITERS = 31
MAX_FINITE_BITS = 0x7F7FFFFF


def _tc_kernel(x_ref, w1_ref, b1_ref, w2_ref, b2_ref,
               sparse_ref, mask_ref, sparsity_ref, asp_ref, l1_ref):
    pid = pl.program_id(0)
    x = x_ref[...]

    h = jnp.maximum(
        jax.lax.dot_general(x.astype(jnp.bfloat16), w1_ref[...],
                            (((1,), (0,)), ((), ())),
                            preferred_element_type=jnp.float32)
        + b1_ref[...], 0.0)
    hb = h.astype(jnp.bfloat16).astype(jnp.float32)
    z = jnp.sum(hb * w2_ref[...].astype(jnp.float32), axis=1,
                keepdims=True) + b2_ref[...]
    s = jax.nn.sigmoid(z)
    sparsity = MIN_S + (MAX_S - MIN_S) * s
    kf = jnp.round(D * (1.0 - sparsity))
    k = jnp.maximum(1, kf.astype(jnp.int32))

    bits = jax.lax.bitcast_convert_type(x, jnp.int32) & jnp.int32(0x7FFFFFFF)

    def body(_, carry):
        lo, hi = carry
        mid = lo + ((hi - lo) >> 1)
        cnt = jnp.sum((bits <= mid).astype(jnp.int32), axis=1, keepdims=True)
        pred = cnt >= k
        return (jnp.where(pred, lo, mid + 1), jnp.where(pred, mid, hi))

    lo = jnp.zeros((BR, 1), jnp.int32)
    hi = jnp.full((BR, 1), MAX_FINITE_BITS, jnp.int32)
    lo, hi = jax.lax.fori_loop(0, SEARCH_ITERS, body, (lo, hi))
    thr_bits = lo

    maskf = (bits > thr_bits).astype(jnp.float32)
    sparse = x * maskf
    sparse_ref[...] = sparse
    mask_ref[...] = maskf
    sparsity_ref[...] = sparsity
    asp_ref[...] = jnp.sum(maskf, axis=1, keepdims=True) * (1.0 / D)

    blk_l1 = jnp.sum(jnp.abs(sparse))

    @pl.when(pid == 0)
    def _():
        l1_ref[...] = jnp.zeros_like(l1_ref)

    l1_ref[...] += blk_l1

    @pl.when(pid == NBLK - 1)
    def _():
        l1_ref[...] = l1_ref[...] * (1.0 / B)


@jax.jit
def kernel(x, W1, b1, W2, b2):
    b1r = b1.reshape(1, H)
    w2r = W2.reshape(1, H).astype(jnp.bfloat16)
    b2r = b2.reshape(1, 1)
    W1 = W1.astype(jnp.bfloat16)
    out_shapes = (
        jax.ShapeDtypeStruct((B, D), jnp.float32),
        jax.ShapeDtypeStruct((B, D), jnp.float32),
        jax.ShapeDtypeStruct((B, 1), jnp.float32),
        jax.ShapeDtypeStruct((B, 1), jnp.float32),
        jax.ShapeDtypeStruct((1, 1), jnp.float32),
    )
    row_spec = pl.BlockSpec((BR, D), lambda i: (i, 0))
    full = lambda shape: pl.BlockSpec(shape, lambda i: (0,) * len(shape))
    sparse_x, mask, sparsity, asp, l1 = pl.pallas_call(
        _tc_kernel,
        grid=(NBLK,),
        in_specs=[row_spec, full((D, H)), full((1, H)), full((1, H)),
                  full((1, 1))],
        out_specs=(row_spec, row_spec,
                   pl.BlockSpec((BR, 1), lambda i: (i, 0)),
                   pl.BlockSpec((BR, 1), lambda i: (i, 0)),
                   full((1, 1))),
        out_shape=out_shapes,
    )(x, W1, b1r, w2r, b2r)
    return sparse_x, mask, sparsity, asp.reshape(B), l1.reshape(())
